# Initial kernel scaffold; baseline (speedup 1.0000x reference)
import jax
import jax.numpy as jnp
from jax.experimental import pallas as pl


def kernel(inputs):
    raise NotImplementedError("write your pallas kernel here")


# speedup vs baseline: 1.4176x; 1.4176x over previous
import jax
import jax.numpy as jnp
from jax.experimental import pallas as pl

UST_DIM = 8


def _nn_scale_kernel(scale_ref):
    F = scale_ref.shape[-1]
    qi = jax.lax.broadcasted_iota(jnp.int32, (F, F), 0)
    pj = jax.lax.broadcasted_iota(jnp.int32, (F, F), 1)
    diff = (qi - pj).astype(jnp.float32)
    dists = jnp.float32(UST_DIM) * (diff * diff)
    idx = jnp.argmin(dists, axis=1)
    scale_ref[...] = ((idx.astype(jnp.float32) + 1.0) / jnp.float32(F))[None, :]


def _mul_kernel(x_ref, scale_ref, o_ref):
    o_ref[...] = x_ref[...] * scale_ref[...]


def kernel(inputs):
    B, F = inputs.shape
    scale = pl.pallas_call(
        _nn_scale_kernel,
        out_shape=jax.ShapeDtypeStruct((1, F), jnp.float32),
    )()
    BLK = 2048
    out = pl.pallas_call(
        _mul_kernel,
        grid=(B // BLK,),
        in_specs=[
            pl.BlockSpec((BLK, F), lambda i: (i, 0)),
            pl.BlockSpec((1, F), lambda i: (0, 0)),
        ],
        out_specs=pl.BlockSpec((BLK, F), lambda i: (i, 0)),
        out_shape=jax.ShapeDtypeStruct((B, F), inputs.dtype),
    )(inputs, scale)
    return out
